# baseline (speedup 1.0000x reference)
import functools

import jax
import jax.numpy as jnp
from jax import lax
from jax.experimental import pallas as pl
from jax.experimental.pallas import tpu as pltpu
from jax.experimental.pallas import tpu_sc as plsc

UNK = 0
IGNORE = -100
EPS = 1e-20
LN2 = 0.6931471805599453
SQRT2 = 1.4142135623730951


def _vlog(x):
    bits = lax.bitcast_convert_type(x, jnp.int32)
    e = ((bits >> 23) & 0xFF) - 127
    m = lax.bitcast_convert_type((bits & 0x007FFFFF) | 0x3F800000, jnp.float32)
    big = m >= SQRT2
    m = jnp.where(big, m * 0.5, m)
    e = jnp.where(big, e + 1, e)
    s = (m - 1.0) / (m + 1.0)
    s2 = s * s
    p = 2.0 * s * (1.0 + s2 * (1.0 / 3.0 + s2 * (0.2 + s2 * (1.0 / 7.0))))
    return e.astype(jnp.float32) * LN2 + p


def kernel(scores, align_src, align_conv, target):
    planes, n, vocab = scores.shape
    scores_flat = scores.reshape(-1)
    align_src = align_src.astype(jnp.int32)
    align_conv = align_conv.astype(jnp.int32)
    target = target.astype(jnp.int32)

    info = plsc.get_sparse_core_info()
    nw = info.num_cores * info.num_subcores
    bw = n // nw
    nch = bw // 16

    mesh = plsc.VectorSubcoreMesh(core_axis_name="c", subcore_axis_name="s")

    @functools.partial(
        pl.kernel,
        mesh=mesh,
        out_type=jax.ShapeDtypeStruct((n,), jnp.float32),
        scratch_types=[
            pltpu.VMEM((bw,), jnp.int32),
            pltpu.VMEM((bw,), jnp.int32),
            pltpu.VMEM((bw,), jnp.int32),
            pltpu.VMEM((3 * bw,), jnp.int32),
            pltpu.VMEM((3 * bw,), jnp.float32),
            pltpu.VMEM((bw,), jnp.float32),
            pltpu.SemaphoreType.DMA,
            pltpu.SemaphoreType.DMA,
        ],
    )
    def _k(scores_hbm, tgt_hbm, src_hbm, conv_hbm, out_hbm,
           tgt_v, src_v, conv_v, idx_v, vals_v, out_v, sem_in, sem_g):
        wid = lax.axis_index("s") * info.num_cores + lax.axis_index("c")
        base = wid * bw
        cp_t = pltpu.make_async_copy(tgt_hbm.at[pl.ds(base, bw)], tgt_v, sem_in)
        cp_s = pltpu.make_async_copy(src_hbm.at[pl.ds(base, bw)], src_v, sem_in)
        cp_c = pltpu.make_async_copy(conv_hbm.at[pl.ds(base, bw)], conv_v, sem_in)
        cp_t.start(); cp_s.start(); cp_c.start()
        cp_t.wait(); cp_s.wait(); cp_c.wait()
        lanes = lax.iota(jnp.int32, 16)
        for j in range(nch):
            row_off = (base + j * 16 + lanes) * vocab
            idx_v[pl.ds(j * 16, 16)] = row_off + tgt_v[pl.ds(j * 16, 16)]
            idx_v[pl.ds(bw + j * 16, 16)] = (n * vocab) + row_off + src_v[pl.ds(j * 16, 16)]
            idx_v[pl.ds(2 * bw + j * 16, 16)] = (2 * n * vocab) + row_off + conv_v[pl.ds(j * 16, 16)]
        pltpu.async_copy(scores_hbm.at[idx_v], vals_v, sem_g).wait()
        for j in range(nch):
            v = vals_v[pl.ds(j * 16, 16)]
            c = vals_v[pl.ds(bw + j * 16, 16)]
            cc = vals_v[pl.ds(2 * bw + j * 16, 16)]
            tgt = tgt_v[pl.ds(j * 16, 16)]
            src = src_v[pl.ds(j * 16, 16)]
            conv = conv_v[pl.ds(j * 16, 16)]
            src_unk = src == UNK
            conv_unk = conv == UNK
            ct = jnp.where(src_unk, 0.0, c) + EPS
            ccv = jnp.where(conv_unk, 0.0, cc) + EPS
            non_copy = (src_unk & conv_unk) | (tgt != UNK)
            probs = ct + ccv + jnp.where(non_copy, v, 0.0)
            loss = -_vlog(probs)
            out_v[pl.ds(j * 16, 16)] = jnp.where(tgt == IGNORE, 0.0, loss)
        pltpu.sync_copy(out_v, out_hbm.at[pl.ds(base, bw)])

    return _k(scores_flat, target, align_src, align_conv)


# speedup vs baseline: 2.3990x; 2.3990x over previous
import functools

import jax
import jax.numpy as jnp
from jax import lax
from jax.experimental import pallas as pl
from jax.experimental.pallas import tpu as pltpu
from jax.experimental.pallas import tpu_sc as plsc

UNK = 0
IGNORE = -100
EPS = 1e-20
LN2 = 0.6931471805599453
SQRT2 = 1.4142135623730951


def _vlog(x):
    bits = lax.bitcast_convert_type(x, jnp.int32)
    e = ((bits >> 23) & 0xFF) - 127
    m = lax.bitcast_convert_type((bits & 0x007FFFFF) | 0x3F800000, jnp.float32)
    big = m >= SQRT2
    m = jnp.where(big, m * 0.5, m)
    e = jnp.where(big, e + 1, e)
    s = (m - 1.0) / (m + 1.0)
    s2 = s * s
    p = 2.0 * s * (1.0 + s2 * (1.0 / 3.0 + s2 * (0.2 + s2 * (1.0 / 7.0))))
    return e.astype(jnp.float32) * LN2 + p


def kernel(scores, align_src, align_conv, target):
    planes, n, vocab = scores.shape
    align_src = align_src.astype(jnp.int32)
    align_conv = align_conv.astype(jnp.int32)
    target = target.astype(jnp.int32)

    info = plsc.get_sparse_core_info()
    nw = info.num_cores * info.num_subcores
    bw = n // nw
    nch = bw // 16

    mesh = plsc.VectorSubcoreMesh(core_axis_name="c", subcore_axis_name="s")

    @functools.partial(
        pl.kernel,
        mesh=mesh,
        compiler_params=pltpu.CompilerParams(needs_layout_passes=False),
        out_type=jax.ShapeDtypeStruct((n,), jnp.float32),
        scratch_types=[
            pltpu.VMEM((bw,), jnp.int32),
            pltpu.VMEM((bw,), jnp.int32),
            pltpu.VMEM((bw,), jnp.int32),
            pltpu.VMEM((3 * bw, 8, 128), jnp.float32),
            pltpu.VMEM((bw,), jnp.float32),
            pltpu.SemaphoreType.DMA,
            pltpu.SemaphoreType.DMA,
        ],
    )
    def _k(scores_hbm, tgt_hbm, src_hbm, conv_hbm, out_hbm,
           tgt_v, src_v, conv_v, win_v, out_v, sem_i, sem_w):
        wid = lax.axis_index("s") * info.num_cores + lax.axis_index("c")
        base = wid * bw
        cp_t = pltpu.make_async_copy(tgt_hbm.at[pl.ds(base, bw)], tgt_v, sem_i)
        cp_s = pltpu.make_async_copy(src_hbm.at[pl.ds(base, bw)], src_v, sem_i)
        cp_c = pltpu.make_async_copy(conv_hbm.at[pl.ds(base, bw)], conv_v, sem_i)
        cp_t.start(); cp_s.start(); cp_c.start()
        cp_t.wait(); cp_s.wait(); cp_c.wait()
        lanes = lax.iota(jnp.int32, 16)
        for p, col_ref in ((0, tgt_v), (1, src_v), (2, conv_v)):
            for jj in range(nch):
                chunk = col_ref[pl.ds(jj * 16, 16)]
                for l in range(16):
                    j = jj * 16 + l
                    c = jnp.max(jnp.where(lanes == l, chunk, -1))
                    cs = pl.multiple_of((c >> 7) << 7, 128)
                    rs = pl.multiple_of(base + (j // 8) * 8, 8)
                    pltpu.make_async_copy(
                        scores_hbm.at[p, pl.ds(rs, 8), pl.ds(cs, 128)],
                        win_v.at[p * bw + j], sem_w,
                    ).start()
        for _ in range(3 * bw):
            pltpu.make_async_copy(
                scores_hbm.at[0, pl.ds(0, 8), pl.ds(0, 128)], win_v.at[0], sem_w
            ).wait()
        for jj in range(nch):
            j16 = jj * 16 + lanes
            rit = j16 & 7
            tgt = tgt_v[pl.ds(jj * 16, 16)]
            src = src_v[pl.ds(jj * 16, 16)]
            conv = conv_v[pl.ds(jj * 16, 16)]
            v = plsc.load_gather(win_v, [j16, rit, tgt & 127])
            c = plsc.load_gather(win_v, [bw + j16, rit, src & 127])
            cc = plsc.load_gather(win_v, [2 * bw + j16, rit, conv & 127])
            src_unk = src == UNK
            conv_unk = conv == UNK
            ct = jnp.where(src_unk, 0.0, c) + EPS
            ccv = jnp.where(conv_unk, 0.0, cc) + EPS
            non_copy = (src_unk & conv_unk) | (tgt != UNK)
            probs = ct + ccv + jnp.where(non_copy, v, 0.0)
            loss = -_vlog(probs)
            out_v[pl.ds(jj * 16, 16)] = jnp.where(tgt == IGNORE, 0.0, loss)
        pltpu.sync_copy(out_v, out_hbm.at[pl.ds(base, bw)])

    return _k(scores, target, align_src, align_conv)


# speedup vs baseline: 93.6190x; 39.0245x over previous
import functools

import jax
import jax.numpy as jnp
from jax import lax
from jax.experimental import pallas as pl
from jax.experimental.pallas import tpu as pltpu
from jax.experimental.pallas import tpu_sc as plsc

UNK = 0
IGNORE = -100
EPS = 1e-20
LN2 = 0.6931471805599453
SQRT2 = 1.4142135623730951


def _vlog(x):
    bits = lax.bitcast_convert_type(x, jnp.int32)
    e = ((bits >> 23) & 0xFF) - 127
    m = lax.bitcast_convert_type((bits & 0x007FFFFF) | 0x3F800000, jnp.float32)
    big = m >= SQRT2
    m = jnp.where(big, m * 0.5, m)
    e = jnp.where(big, e + 1, e)
    s = (m - 1.0) / (m + 1.0)
    s2 = s * s
    p = 2.0 * s * (1.0 + s2 * (1.0 / 3.0 + s2 * (0.2 + s2 * (1.0 / 7.0))))
    return e.astype(jnp.float32) * LN2 + p


def kernel(scores, align_src, align_conv, target):
    planes, n, vocab = scores.shape
    scores_t = jnp.transpose(scores, (0, 2, 1))
    align_src = align_src.astype(jnp.int32)
    align_conv = align_conv.astype(jnp.int32)
    target = target.astype(jnp.int32)

    info = plsc.get_sparse_core_info()
    nw = info.num_cores * info.num_subcores
    bw = n // nw
    nch = bw // 16

    mesh = plsc.VectorSubcoreMesh(core_axis_name="c", subcore_axis_name="s")

    @functools.partial(
        pl.kernel,
        mesh=mesh,
        compiler_params=pltpu.CompilerParams(needs_layout_passes=False),
        out_type=jax.ShapeDtypeStruct((n,), jnp.float32),
        scratch_types=[
            pltpu.VMEM((bw,), jnp.int32),
            pltpu.VMEM((bw,), jnp.int32),
            pltpu.VMEM((bw,), jnp.int32),
            pltpu.VMEM((3 * bw, 8, 128), jnp.float32),
            pltpu.VMEM((bw,), jnp.float32),
            pltpu.SemaphoreType.DMA,
            pltpu.SemaphoreType.DMA,
        ],
    )
    def _k(scores_hbm, tgt_hbm, src_hbm, conv_hbm, out_hbm,
           tgt_v, src_v, conv_v, win_v, out_v, sem_i, sem_w):
        wid = lax.axis_index("s") * info.num_cores + lax.axis_index("c")
        base = wid * bw
        cp_t = pltpu.make_async_copy(tgt_hbm.at[pl.ds(base, bw)], tgt_v, sem_i)
        cp_s = pltpu.make_async_copy(src_hbm.at[pl.ds(base, bw)], src_v, sem_i)
        cp_c = pltpu.make_async_copy(conv_hbm.at[pl.ds(base, bw)], conv_v, sem_i)
        cp_t.start(); cp_s.start(); cp_c.start()
        cp_t.wait(); cp_s.wait(); cp_c.wait()
        lanes = lax.iota(jnp.int32, 16)
        rs = pl.multiple_of((base >> 7) << 7, 128)
        rin = (base & 127) + lanes
        for p, col_ref in ((0, tgt_v), (1, src_v), (2, conv_v)):
            for jj in range(nch):
                chunk = col_ref[pl.ds(jj * 16, 16)]
                for l in range(16):
                    j = jj * 16 + l
                    c = jnp.max(jnp.where(lanes == l, chunk, -1))
                    cs = pl.multiple_of((c >> 3) << 3, 8)
                    pltpu.make_async_copy(
                        scores_hbm.at[p, pl.ds(cs, 8), pl.ds(rs, 128)],
                        win_v.at[p * bw + j], sem_w,
                    ).start()
        for _ in range(3 * bw):
            pltpu.make_async_copy(
                scores_hbm.at[0, pl.ds(0, 8), pl.ds(0, 128)], win_v.at[0], sem_w
            ).wait()
        for jj in range(nch):
            j16 = jj * 16 + lanes
            rloc = rin + jj * 16
            tgt = tgt_v[pl.ds(jj * 16, 16)]
            src = src_v[pl.ds(jj * 16, 16)]
            conv = conv_v[pl.ds(jj * 16, 16)]
            v = plsc.load_gather(win_v, [j16, tgt & 7, rloc])
            c = plsc.load_gather(win_v, [bw + j16, src & 7, rloc])
            cc = plsc.load_gather(win_v, [2 * bw + j16, conv & 7, rloc])
            src_unk = src == UNK
            conv_unk = conv == UNK
            ct = jnp.where(src_unk, 0.0, c) + EPS
            ccv = jnp.where(conv_unk, 0.0, cc) + EPS
            non_copy = (src_unk & conv_unk) | (tgt != UNK)
            probs = ct + ccv + jnp.where(non_copy, v, 0.0)
            loss = -_vlog(probs)
            out_v[pl.ds(jj * 16, 16)] = jnp.where(tgt == IGNORE, 0.0, loss)
        pltpu.sync_copy(out_v, out_hbm.at[pl.ds(base, bw)])

    return _k(scores_t, target, align_src, align_conv)


# speedup vs baseline: 102.6694x; 1.0967x over previous
import functools

import jax
import jax.numpy as jnp
from jax import lax
from jax.experimental import pallas as pl
from jax.experimental.pallas import tpu as pltpu
from jax.experimental.pallas import tpu_sc as plsc

UNK = 0
IGNORE = -100
EPS = 1e-20
LN2 = 0.6931471805599453
SQRT2 = 1.4142135623730951


def _vlog(x):
    bits = lax.bitcast_convert_type(x, jnp.int32)
    e = ((bits >> 23) & 0xFF) - 127
    m = lax.bitcast_convert_type((bits & 0x007FFFFF) | 0x3F800000, jnp.float32)
    big = m >= SQRT2
    m = jnp.where(big, m * 0.5, m)
    e = jnp.where(big, e + 1, e)
    s = (m - 1.0) / (m + 1.0)
    s2 = s * s
    p = 2.0 * s * (1.0 + s2 * (1.0 / 3.0 + s2 * (0.2 + s2 * (1.0 / 7.0))))
    return e.astype(jnp.float32) * LN2 + p


def kernel(scores, align_src, align_conv, target):
    planes, n, vocab = scores.shape
    rows_per_plane = vocab * n // 128
    scores_t = jnp.transpose(scores, (0, 2, 1))
    align_src = align_src.astype(jnp.int32)
    align_conv = align_conv.astype(jnp.int32)
    target = target.astype(jnp.int32)

    info = plsc.get_sparse_core_info()
    nw = info.num_cores * info.num_subcores
    bw = n // nw
    nch = bw // 16

    mesh = plsc.VectorSubcoreMesh(core_axis_name="c", subcore_axis_name="s")

    @functools.partial(
        pl.kernel,
        mesh=mesh,
        compiler_params=pltpu.CompilerParams(needs_layout_passes=False),
        out_type=jax.ShapeDtypeStruct((n,), jnp.float32),
        scratch_types=[
            pltpu.VMEM((bw,), jnp.int32),
            pltpu.VMEM((bw,), jnp.int32),
            pltpu.VMEM((bw,), jnp.int32),
            pltpu.VMEM((3 * bw,), jnp.int32),
            pltpu.VMEM((3 * bw, 1024), jnp.float32),
            pltpu.VMEM((bw,), jnp.float32),
            pltpu.SemaphoreType.DMA,
            pltpu.SemaphoreType.DMA,
        ],
    )
    def _k(scores_hbm, tgt_hbm, src_hbm, conv_hbm, out_hbm,
           tgt_v, src_v, conv_v, idx_v, win_v, out_v, sem_i, sem_w):
        wid = lax.axis_index("s") * info.num_cores + lax.axis_index("c")
        base = wid * bw
        tab = scores_hbm.reshape(planes * vocab, n)
        cp_t = pltpu.make_async_copy(tgt_hbm.at[pl.ds(base, bw)], tgt_v, sem_i)
        cp_s = pltpu.make_async_copy(src_hbm.at[pl.ds(base, bw)], src_v, sem_i)
        cp_c = pltpu.make_async_copy(conv_hbm.at[pl.ds(base, bw)], conv_v, sem_i)
        cp_t.start(); cp_s.start(); cp_c.start()
        cp_t.wait(); cp_s.wait(); cp_c.wait()
        lanes = lax.iota(jnp.int32, 16)
        for jj in range(nch):
            idx_v[pl.ds(jj * 16, 16)] = tgt_v[pl.ds(jj * 16, 16)]
            idx_v[pl.ds(bw + jj * 16, 16)] = vocab + src_v[pl.ds(jj * 16, 16)]
            idx_v[pl.ds(2 * bw + jj * 16, 16)] = 2 * vocab + conv_v[pl.ds(jj * 16, 16)]
        pltpu.async_copy(tab.at[idx_v], win_v, sem_w).wait()
        for jj in range(nch):
            j16 = jj * 16 + lanes
            rloc = base + j16
            tgt = tgt_v[pl.ds(jj * 16, 16)]
            src = src_v[pl.ds(jj * 16, 16)]
            conv = conv_v[pl.ds(jj * 16, 16)]
            v = plsc.load_gather(win_v, [j16, rloc])
            c = plsc.load_gather(win_v, [bw + j16, rloc])
            cc = plsc.load_gather(win_v, [2 * bw + j16, rloc])
            src_unk = src == UNK
            conv_unk = conv == UNK
            ct = jnp.where(src_unk, 0.0, c) + EPS
            ccv = jnp.where(conv_unk, 0.0, cc) + EPS
            non_copy = (src_unk & conv_unk) | (tgt != UNK)
            probs = ct + ccv + jnp.where(non_copy, v, 0.0)
            loss = -_vlog(probs)
            out_v[pl.ds(jj * 16, 16)] = jnp.where(tgt == IGNORE, 0.0, loss)
        pltpu.sync_copy(out_v, out_hbm.at[pl.ds(base, bw)])

    return _k(scores_t, target, align_src, align_conv)


# speedup vs baseline: 118.3664x; 1.1529x over previous
import functools

import jax
import jax.numpy as jnp
from jax import lax
from jax.experimental import pallas as pl
from jax.experimental.pallas import tpu as pltpu
from jax.experimental.pallas import tpu_sc as plsc

UNK = 0
IGNORE = -100
EPS = 1e-20
LN2 = 0.6931471805599453
SQRT2 = 1.4142135623730951


def _vlog(x):
    bits = lax.bitcast_convert_type(x, jnp.int32)
    e = ((bits >> 23) & 0xFF) - 127
    m = lax.bitcast_convert_type((bits & 0x007FFFFF) | 0x3F800000, jnp.float32)
    big = m >= SQRT2
    m = jnp.where(big, m * 0.5, m)
    e = jnp.where(big, e + 1, e)
    s = (m - 1.0) / (m + 1.0)
    s2 = s * s
    p = 2.0 * s * (1.0 + s2 * (1.0 / 3.0 + s2 * (0.2 + s2 * (1.0 / 7.0))))
    return e.astype(jnp.float32) * LN2 + p


def kernel(scores, align_src, align_conv, target):
    planes, n, vocab = scores.shape
    rows_per_plane = vocab * n // 128
    scores_t = jnp.transpose(scores, (0, 2, 1))
    align_src = align_src.astype(jnp.int32)
    align_conv = align_conv.astype(jnp.int32)
    target = target.astype(jnp.int32)

    info = plsc.get_sparse_core_info()
    nw = info.num_cores * info.num_subcores
    bw = n // nw
    nch = bw // 16

    mesh = plsc.VectorSubcoreMesh(core_axis_name="c", subcore_axis_name="s")

    @functools.partial(
        pl.kernel,
        mesh=mesh,
        compiler_params=pltpu.CompilerParams(needs_layout_passes=False),
        out_type=jax.ShapeDtypeStruct((n,), jnp.float32),
        scratch_types=[
            pltpu.VMEM((bw,), jnp.int32),
            pltpu.VMEM((bw,), jnp.int32),
            pltpu.VMEM((bw,), jnp.int32),
            pltpu.VMEM((3 * bw,), jnp.int32),
            pltpu.VMEM((3 * bw, 128), jnp.float32),
            pltpu.VMEM((bw,), jnp.float32),
            pltpu.SemaphoreType.DMA,
            pltpu.SemaphoreType.DMA,
        ],
    )
    def _k(scores_hbm, tgt_hbm, src_hbm, conv_hbm, out_hbm,
           tgt_v, src_v, conv_v, idx_v, win_v, out_v, sem_i, sem_w):
        wid = lax.axis_index("s") * info.num_cores + lax.axis_index("c")
        base = wid * bw
        tab = scores_hbm.reshape(planes * vocab, n)
        cp_t = pltpu.make_async_copy(tgt_hbm.at[pl.ds(base, bw)], tgt_v, sem_i)
        cp_s = pltpu.make_async_copy(src_hbm.at[pl.ds(base, bw)], src_v, sem_i)
        cp_c = pltpu.make_async_copy(conv_hbm.at[pl.ds(base, bw)], conv_v, sem_i)
        cp_t.start(); cp_s.start(); cp_c.start()
        cp_t.wait(); cp_s.wait(); cp_c.wait()
        lanes = lax.iota(jnp.int32, 16)
        for jj in range(nch):
            idx_v[pl.ds(jj * 16, 16)] = tgt_v[pl.ds(jj * 16, 16)]
            idx_v[pl.ds(bw + jj * 16, 16)] = vocab + src_v[pl.ds(jj * 16, 16)]
            idx_v[pl.ds(2 * bw + jj * 16, 16)] = 2 * vocab + conv_v[pl.ds(jj * 16, 16)]
        rs = pl.multiple_of((base >> 7) << 7, 128)
        pltpu.async_copy(tab.at[idx_v, pl.ds(rs, 128)], win_v, sem_w).wait()
        for jj in range(nch):
            j16 = jj * 16 + lanes
            rloc = (base & 127) + j16
            tgt = tgt_v[pl.ds(jj * 16, 16)]
            src = src_v[pl.ds(jj * 16, 16)]
            conv = conv_v[pl.ds(jj * 16, 16)]
            v = plsc.load_gather(win_v, [j16, rloc])
            c = plsc.load_gather(win_v, [bw + j16, rloc])
            cc = plsc.load_gather(win_v, [2 * bw + j16, rloc])
            src_unk = src == UNK
            conv_unk = conv == UNK
            ct = jnp.where(src_unk, 0.0, c) + EPS
            ccv = jnp.where(conv_unk, 0.0, cc) + EPS
            non_copy = (src_unk & conv_unk) | (tgt != UNK)
            probs = ct + ccv + jnp.where(non_copy, v, 0.0)
            loss = -_vlog(probs)
            out_v[pl.ds(jj * 16, 16)] = jnp.where(tgt == IGNORE, 0.0, loss)
        pltpu.sync_copy(out_v, out_hbm.at[pl.ds(base, bw)])

    return _k(scores_t, target, align_src, align_conv)


# speedup vs baseline: 118.5986x; 1.0020x over previous
import functools

import jax
import jax.numpy as jnp
from jax import lax
from jax.experimental import pallas as pl
from jax.experimental.pallas import tpu as pltpu
from jax.experimental.pallas import tpu_sc as plsc

UNK = 0
IGNORE = -100
EPS = 1e-20
LN2 = 0.6931471805599453
SQRT2 = 1.4142135623730951


def _vlog(x):
    bits = lax.bitcast_convert_type(x, jnp.int32)
    e = ((bits >> 23) & 0xFF) - 127
    m = lax.bitcast_convert_type((bits & 0x007FFFFF) | 0x3F800000, jnp.float32)
    big = m >= SQRT2
    m = jnp.where(big, m * 0.5, m)
    e = jnp.where(big, e + 1, e)
    s = (m - 1.0) / (m + 1.0)
    s2 = s * s
    p = 2.0 * s * (1.0 + s2 * (1.0 / 3.0 + s2 * (0.2 + s2 * (1.0 / 7.0))))
    return e.astype(jnp.float32) * LN2 + p


def kernel(scores, align_src, align_conv, target):
    planes, n, vocab = scores.shape
    rows_per_plane = vocab * n // 128
    scores_t = jnp.transpose(scores, (0, 2, 1))
    align_src = align_src.astype(jnp.int32)
    align_conv = align_conv.astype(jnp.int32)
    target = target.astype(jnp.int32)

    info = plsc.get_sparse_core_info()
    nw = info.num_cores * info.num_subcores
    bw = n // nw
    nch = bw // 16

    mesh = plsc.VectorSubcoreMesh(core_axis_name="c", subcore_axis_name="s")

    @functools.partial(
        pl.kernel,
        mesh=mesh,
        compiler_params=pltpu.CompilerParams(
            needs_layout_passes=False,
            skip_device_barrier=True,
            disable_bounds_checks=True,
            disable_semaphore_checks=True,
        ),
        out_type=jax.ShapeDtypeStruct((n,), jnp.float32),
        scratch_types=[
            pltpu.VMEM((bw,), jnp.int32),
            pltpu.VMEM((bw,), jnp.int32),
            pltpu.VMEM((bw,), jnp.int32),
            pltpu.VMEM((3 * bw,), jnp.int32),
            pltpu.VMEM((3 * bw, 128), jnp.float32),
            pltpu.VMEM((bw,), jnp.float32),
            pltpu.SemaphoreType.DMA,
            pltpu.SemaphoreType.DMA,
        ],
    )
    def _k(scores_hbm, tgt_hbm, src_hbm, conv_hbm, out_hbm,
           tgt_v, src_v, conv_v, idx_v, win_v, out_v, sem_i, sem_w):
        wid = lax.axis_index("s") * info.num_cores + lax.axis_index("c")
        base = wid * bw
        tab = scores_hbm.reshape(planes * vocab, n)
        cp_t = pltpu.make_async_copy(tgt_hbm.at[pl.ds(base, bw)], tgt_v, sem_i)
        cp_s = pltpu.make_async_copy(src_hbm.at[pl.ds(base, bw)], src_v, sem_i)
        cp_c = pltpu.make_async_copy(conv_hbm.at[pl.ds(base, bw)], conv_v, sem_i)
        cp_t.start(); cp_s.start(); cp_c.start()
        cp_t.wait(); cp_s.wait(); cp_c.wait()
        lanes = lax.iota(jnp.int32, 16)
        for jj in range(nch):
            idx_v[pl.ds(jj * 16, 16)] = tgt_v[pl.ds(jj * 16, 16)]
            idx_v[pl.ds(bw + jj * 16, 16)] = vocab + src_v[pl.ds(jj * 16, 16)]
            idx_v[pl.ds(2 * bw + jj * 16, 16)] = 2 * vocab + conv_v[pl.ds(jj * 16, 16)]
        rs = pl.multiple_of((base >> 7) << 7, 128)
        pltpu.async_copy(tab.at[idx_v, pl.ds(rs, 128)], win_v, sem_w).wait()
        for jj in range(nch):
            j16 = jj * 16 + lanes
            rloc = (base & 127) + j16
            tgt = tgt_v[pl.ds(jj * 16, 16)]
            src = src_v[pl.ds(jj * 16, 16)]
            conv = conv_v[pl.ds(jj * 16, 16)]
            v = plsc.load_gather(win_v, [j16, rloc])
            c = plsc.load_gather(win_v, [bw + j16, rloc])
            cc = plsc.load_gather(win_v, [2 * bw + j16, rloc])
            src_unk = src == UNK
            conv_unk = conv == UNK
            ct = jnp.where(src_unk, 0.0, c) + EPS
            ccv = jnp.where(conv_unk, 0.0, cc) + EPS
            non_copy = (src_unk & conv_unk) | (tgt != UNK)
            probs = ct + ccv + jnp.where(non_copy, v, 0.0)
            loss = -_vlog(probs)
            out_v[pl.ds(jj * 16, 16)] = jnp.where(tgt == IGNORE, 0.0, loss)
        pltpu.sync_copy(out_v, out_hbm.at[pl.ds(base, bw)])

    return _k(scores_t, target, align_src, align_conv)


# speedup vs baseline: 118.6412x; 1.0004x over previous
import functools

import jax
import jax.numpy as jnp
from jax import lax
from jax.experimental import pallas as pl
from jax.experimental.pallas import tpu as pltpu
from jax.experimental.pallas import tpu_sc as plsc

UNK = 0
IGNORE = -100
EPS = 1e-20
LN2 = 0.6931471805599453
SQRT2 = 1.4142135623730951


def _vlog(x):
    bits = lax.bitcast_convert_type(x, jnp.int32)
    e = ((bits >> 23) & 0xFF) - 127
    m = lax.bitcast_convert_type((bits & 0x007FFFFF) | 0x3F800000, jnp.float32)
    big = m >= SQRT2
    m = jnp.where(big, m * 0.5, m)
    e = jnp.where(big, e + 1, e)
    s = (m - 1.0) / (m + 1.0)
    s2 = s * s
    p = 2.0 * s * (1.0 + s2 * (1.0 / 3.0 + s2 * (0.2 + s2 * (1.0 / 7.0))))
    return e.astype(jnp.float32) * LN2 + p


def kernel(scores, align_src, align_conv, target):
    planes, n, vocab = scores.shape
    rows_per_plane = vocab * n // 128
    scores_t = jnp.transpose(scores, (0, 2, 1))
    align_src = align_src.astype(jnp.int32)
    align_conv = align_conv.astype(jnp.int32)
    target = target.astype(jnp.int32)

    info = plsc.get_sparse_core_info()
    nw = info.num_cores * info.num_subcores
    bw = n // nw
    nch = bw // 16

    mesh = plsc.VectorSubcoreMesh(core_axis_name="c", subcore_axis_name="s")

    @functools.partial(
        pl.kernel,
        mesh=mesh,
        compiler_params=pltpu.CompilerParams(needs_layout_passes=False),
        out_type=jax.ShapeDtypeStruct((n,), jnp.float32),
        scratch_types=[
            pltpu.VMEM((bw,), jnp.int32),
            pltpu.VMEM((bw,), jnp.int32),
            pltpu.VMEM((bw,), jnp.int32),
            pltpu.VMEM((3 * bw,), jnp.int32),
            pltpu.VMEM((3 * bw, 128), jnp.float32),
            pltpu.VMEM((bw,), jnp.float32),
            pltpu.SemaphoreType.DMA,
            pltpu.SemaphoreType.DMA,
        ],
    )
    def _k(scores_hbm, tgt_hbm, src_hbm, conv_hbm, out_hbm,
           tgt_v, src_v, conv_v, idx_v, win_v, out_v, sem_i, sem_w):
        wid = lax.axis_index("s") * info.num_cores + lax.axis_index("c")
        base = wid * bw
        tab = scores_hbm.reshape(planes * vocab, n)
        cp_t = pltpu.make_async_copy(tgt_hbm.at[pl.ds(base, bw)], tgt_v, sem_i)
        cp_s = pltpu.make_async_copy(src_hbm.at[pl.ds(base, bw)], src_v, sem_i)
        cp_c = pltpu.make_async_copy(conv_hbm.at[pl.ds(base, bw)], conv_v, sem_i)
        cp_t.start(); cp_s.start(); cp_c.start()
        cp_t.wait(); cp_s.wait(); cp_c.wait()
        lanes = lax.iota(jnp.int32, 16)
        for jj in range(nch):
            idx_v[pl.ds(jj * 16, 16)] = tgt_v[pl.ds(jj * 16, 16)]
            idx_v[pl.ds(bw + jj * 16, 16)] = vocab + src_v[pl.ds(jj * 16, 16)]
            idx_v[pl.ds(2 * bw + jj * 16, 16)] = 2 * vocab + conv_v[pl.ds(jj * 16, 16)]
        rs = pl.multiple_of((base >> 7) << 7, 128)
        pltpu.async_copy(tab.at[idx_v, pl.ds(rs, 128)], win_v, sem_w).wait()
        for jj in range(nch):
            j16 = jj * 16 + lanes
            rloc = (base & 127) + j16
            tgt = tgt_v[pl.ds(jj * 16, 16)]
            src = src_v[pl.ds(jj * 16, 16)]
            conv = conv_v[pl.ds(jj * 16, 16)]
            v = plsc.load_gather(win_v, [j16, rloc])
            c = plsc.load_gather(win_v, [bw + j16, rloc])
            cc = plsc.load_gather(win_v, [2 * bw + j16, rloc])
            src_unk = src == UNK
            conv_unk = conv == UNK
            ct = jnp.where(src_unk, 0.0, c) + EPS
            ccv = jnp.where(conv_unk, 0.0, cc) + EPS
            non_copy = (src_unk & conv_unk) | (tgt != UNK)
            probs = ct + ccv + jnp.where(non_copy, v, 0.0)
            loss = -_vlog(probs)
            out_v[pl.ds(jj * 16, 16)] = jnp.where(tgt == IGNORE, 0.0, loss)
        pltpu.sync_copy(out_v, out_hbm.at[pl.ds(base, bw)])

    return _k(scores_t, target, align_src, align_conv)


# speedup vs baseline: 120.0103x; 1.0115x over previous
import functools

import jax
import jax.numpy as jnp
from jax import lax
from jax.experimental import pallas as pl
from jax.experimental.pallas import tpu as pltpu
from jax.experimental.pallas import tpu_sc as plsc

UNK = 0
IGNORE = -100
EPS = 1e-20
LN2 = 0.6931471805599453
SQRT2 = 1.4142135623730951


def _vlog(x):
    bits = lax.bitcast_convert_type(x, jnp.int32)
    e = ((bits >> 23) & 0xFF) - 127
    m = lax.bitcast_convert_type((bits & 0x007FFFFF) | 0x3F800000, jnp.float32)
    big = m >= SQRT2
    m = jnp.where(big, m * 0.5, m)
    e = jnp.where(big, e + 1, e)
    s = (m - 1.0) / (m + 1.0)
    s2 = s * s
    p = 2.0 * s * (1.0 + s2 * (1.0 / 3.0 + s2 * (0.2 + s2 * (1.0 / 7.0))))
    return e.astype(jnp.float32) * LN2 + p


def kernel(scores, align_src, align_conv, target):
    planes, n, vocab = scores.shape
    rows_per_plane = vocab * n // 128
    scores_t = jnp.transpose(scores, (0, 2, 1))
    align_src = align_src.astype(jnp.int32)
    align_conv = align_conv.astype(jnp.int32)
    target = target.astype(jnp.int32)

    info = plsc.get_sparse_core_info()
    nc = 1
    nw = nc * info.num_subcores
    bw = n // nw
    nch = bw // 16

    mesh = plsc.VectorSubcoreMesh(core_axis_name="c", subcore_axis_name="s",
                                  num_cores=1)

    @functools.partial(
        pl.kernel,
        mesh=mesh,
        compiler_params=pltpu.CompilerParams(needs_layout_passes=False),
        out_type=jax.ShapeDtypeStruct((n,), jnp.float32),
        scratch_types=[
            pltpu.VMEM((bw,), jnp.int32),
            pltpu.VMEM((bw,), jnp.int32),
            pltpu.VMEM((bw,), jnp.int32),
            pltpu.VMEM((3 * bw,), jnp.int32),
            pltpu.VMEM((3 * bw, 128), jnp.float32),
            pltpu.VMEM((bw,), jnp.float32),
            pltpu.SemaphoreType.DMA,
            pltpu.SemaphoreType.DMA,
        ],
    )
    def _k(scores_hbm, tgt_hbm, src_hbm, conv_hbm, out_hbm,
           tgt_v, src_v, conv_v, idx_v, win_v, out_v, sem_i, sem_w):
        wid = lax.axis_index("s") * nc + lax.axis_index("c")
        base = wid * bw
        tab = scores_hbm.reshape(planes * vocab, n)
        cp_t = pltpu.make_async_copy(tgt_hbm.at[pl.ds(base, bw)], tgt_v, sem_i)
        cp_s = pltpu.make_async_copy(src_hbm.at[pl.ds(base, bw)], src_v, sem_i)
        cp_c = pltpu.make_async_copy(conv_hbm.at[pl.ds(base, bw)], conv_v, sem_i)
        cp_t.start(); cp_s.start(); cp_c.start()
        cp_t.wait(); cp_s.wait(); cp_c.wait()
        lanes = lax.iota(jnp.int32, 16)
        for jj in range(nch):
            idx_v[pl.ds(jj * 16, 16)] = tgt_v[pl.ds(jj * 16, 16)]
            idx_v[pl.ds(bw + jj * 16, 16)] = vocab + src_v[pl.ds(jj * 16, 16)]
            idx_v[pl.ds(2 * bw + jj * 16, 16)] = 2 * vocab + conv_v[pl.ds(jj * 16, 16)]
        rs = pl.multiple_of((base >> 7) << 7, 128)
        pltpu.async_copy(tab.at[idx_v, pl.ds(rs, 128)], win_v, sem_w).wait()
        for jj in range(nch):
            j16 = jj * 16 + lanes
            rloc = (base & 127) + j16
            tgt = tgt_v[pl.ds(jj * 16, 16)]
            src = src_v[pl.ds(jj * 16, 16)]
            conv = conv_v[pl.ds(jj * 16, 16)]
            v = plsc.load_gather(win_v, [j16, rloc])
            c = plsc.load_gather(win_v, [bw + j16, rloc])
            cc = plsc.load_gather(win_v, [2 * bw + j16, rloc])
            src_unk = src == UNK
            conv_unk = conv == UNK
            ct = jnp.where(src_unk, 0.0, c) + EPS
            ccv = jnp.where(conv_unk, 0.0, cc) + EPS
            non_copy = (src_unk & conv_unk) | (tgt != UNK)
            probs = ct + ccv + jnp.where(non_copy, v, 0.0)
            loss = -_vlog(probs)
            out_v[pl.ds(jj * 16, 16)] = jnp.where(tgt == IGNORE, 0.0, loss)
        pltpu.sync_copy(out_v, out_hbm.at[pl.ds(base, bw)])

    return _k(scores_t, target, align_src, align_conv)


# speedup vs baseline: 120.4341x; 1.0035x over previous
import functools

import jax
import jax.numpy as jnp
from jax import lax
from jax.experimental import pallas as pl
from jax.experimental.pallas import tpu as pltpu
from jax.experimental.pallas import tpu_sc as plsc

UNK = 0
IGNORE = -100
EPS = 1e-20
LN2 = 0.6931471805599453
SQRT2 = 1.4142135623730951


def _vlog(x):
    bits = lax.bitcast_convert_type(x, jnp.int32)
    e = ((bits >> 23) & 0xFF) - 127
    m = lax.bitcast_convert_type((bits & 0x007FFFFF) | 0x3F800000, jnp.float32)
    big = m >= SQRT2
    m = jnp.where(big, m * 0.5, m)
    e = jnp.where(big, e + 1, e)
    s = (m - 1.0) / (m + 1.0)
    s2 = s * s
    p = 2.0 * s * (1.0 + s2 * (1.0 / 3.0 + s2 * (0.2 + s2 * (1.0 / 7.0))))
    return e.astype(jnp.float32) * LN2 + p


def kernel(scores, align_src, align_conv, target):
    planes, n, vocab = scores.shape
    rows_per_plane = vocab * n // 128
    scores_t = jnp.transpose(scores, (0, 2, 1))
    align_src = align_src.astype(jnp.int32)
    align_conv = align_conv.astype(jnp.int32)
    target = target.astype(jnp.int32)

    info = plsc.get_sparse_core_info()
    nc = 1
    nw = nc * info.num_subcores
    bw = n // nw
    nch = bw // 16

    mesh = plsc.VectorSubcoreMesh(core_axis_name="c", subcore_axis_name="s",
                                  num_cores=1)

    @functools.partial(
        pl.kernel,
        mesh=mesh,
        compiler_params=pltpu.CompilerParams(needs_layout_passes=False),
        out_type=jax.ShapeDtypeStruct((n,), jnp.float32),
        scratch_types=[
            pltpu.VMEM((bw,), jnp.int32),
            pltpu.VMEM((bw,), jnp.int32),
            pltpu.VMEM((bw,), jnp.int32),
            pltpu.VMEM((3 * bw,), jnp.int32),
            pltpu.VMEM((3 * bw, 128), jnp.float32),
            pltpu.VMEM((bw,), jnp.float32),
            pltpu.SemaphoreType.DMA,
            pltpu.SemaphoreType.DMA,
        ],
    )
    def _k(scores_hbm, tgt_hbm, src_hbm, conv_hbm, out_hbm,
           tgt_v, src_v, conv_v, idx_v, win_v, out_v, sem_i, sem_w):
        wid = lax.axis_index("s") * nc + lax.axis_index("c")
        base = wid * bw
        tab = scores_hbm.reshape(planes * vocab, n)
        cp_t = pltpu.make_async_copy(tgt_hbm.at[pl.ds(base, bw)], tgt_v, sem_i)
        cp_s = pltpu.make_async_copy(src_hbm.at[pl.ds(base, bw)], src_v, sem_i)
        cp_c = pltpu.make_async_copy(conv_hbm.at[pl.ds(base, bw)], conv_v, sem_i)
        cp_t.start(); cp_s.start(); cp_c.start()
        cp_t.wait(); cp_s.wait(); cp_c.wait()
        lanes = lax.iota(jnp.int32, 16)

        def _build(jj, _):
            o = jj * 16
            idx_v[pl.ds(o, 16)] = tgt_v[pl.ds(o, 16)]
            idx_v[pl.ds(bw + o, 16)] = vocab + src_v[pl.ds(o, 16)]
            idx_v[pl.ds(2 * bw + o, 16)] = 2 * vocab + conv_v[pl.ds(o, 16)]
            return 0

        lax.fori_loop(0, nch, _build, 0)
        rs = pl.multiple_of((base >> 7) << 7, 128)
        pltpu.async_copy(tab.at[idx_v, pl.ds(rs, 128)], win_v, sem_w).wait()

        def _extract(jj, _):
            o = jj * 16
            j16 = o + lanes
            rloc = (base & 127) + j16
            tgt = tgt_v[pl.ds(o, 16)]
            src = src_v[pl.ds(o, 16)]
            conv = conv_v[pl.ds(o, 16)]
            v = plsc.load_gather(win_v, [j16, rloc])
            c = plsc.load_gather(win_v, [bw + j16, rloc])
            cc = plsc.load_gather(win_v, [2 * bw + j16, rloc])
            src_unk = src == UNK
            conv_unk = conv == UNK
            ct = jnp.where(src_unk, 0.0, c) + EPS
            ccv = jnp.where(conv_unk, 0.0, cc) + EPS
            non_copy = (src_unk & conv_unk) | (tgt != UNK)
            probs = ct + ccv + jnp.where(non_copy, v, 0.0)
            loss = -_vlog(probs)
            out_v[pl.ds(o, 16)] = jnp.where(tgt == IGNORE, 0.0, loss)
            return 0

        lax.fori_loop(0, nch, _extract, 0)
        pltpu.sync_copy(out_v, out_hbm.at[pl.ds(base, bw)])

    return _k(scores_t, target, align_src, align_conv)
